# baseline (speedup 1.0000x reference)
import functools

import jax
import jax.numpy as jnp
from jax import lax
from jax.experimental import pallas as pl
from jax.experimental.pallas import tpu as pltpu
from jax.experimental.pallas import tpu_sc as plsc

BATCH = 1024
SEQ = 77
VOCAB = 49408
DIM = 768

NC = 2
NS = 16
NW = NC * NS

B = BATCH * SEQ
B_PER_W = B // NW
CHUNK = 8
G = 7
ROUNDS = B_PER_W // (CHUNK * G)

_mesh = plsc.VectorSubcoreMesh(
    core_axis_name="c", subcore_axis_name="s", num_cores=NC, num_subcores=NS
)


@functools.partial(
    pl.kernel,
    mesh=_mesh,
    out_type=jax.ShapeDtypeStruct((B, DIM), jnp.float32),
    scratch_types=[
        pltpu.VMEM((B_PER_W,), jnp.int32),
        pltpu.VMEM((2, G, CHUNK, DIM), jnp.float32),
        pltpu.SemaphoreType.DMA,
        pltpu.SemaphoreType.DMA,
        pltpu.SemaphoreType.DMA,
        pltpu.SemaphoreType.DMA,
    ],
)
def _sc_gather(idx_hbm, table_hbm, out_hbm, idx_v, bufs, gsem0, gsem1, ssem0, ssem1):
    wid = lax.axis_index("s") * NC + lax.axis_index("c")
    base = wid * B_PER_W
    gsem = (gsem0, gsem1)
    ssem = (ssem0, ssem1)

    pltpu.sync_copy(idx_hbm.at[pl.ds(base, B_PER_W)], idx_v)

    def gathers_start(r, p):
        for j in range(G):
            c = r * G + j
            pltpu.async_copy(
                table_hbm.at[idx_v.at[pl.ds(c * CHUNK, CHUNK)]],
                bufs.at[p, j],
                gsem[p],
            )

    def gathers_wait(p):
        for j in range(G):
            pltpu.make_async_copy(
                table_hbm.at[idx_v.at[pl.ds(0, CHUNK)]], bufs.at[p, j], gsem[p]
            ).wait()

    def stores_start(r, p):
        for j in range(G):
            c = r * G + j
            pltpu.async_copy(
                bufs.at[p, j], out_hbm.at[pl.ds(base + c * CHUNK, CHUNK)], ssem[p]
            )

    def stores_wait(p):
        for j in range(G):
            pltpu.make_async_copy(
                bufs.at[p, j], out_hbm.at[pl.ds(base, CHUNK)], ssem[p]
            ).wait()

    gathers_start(0, 0)
    gathers_wait(0)
    stores_start(0, 0)
    gathers_start(1, 1)

    def body(i, _):
        for p, off in ((1, 1), (0, 2)):
            r = 2 * i + off
            gathers_wait(p)
            stores_start(r, p)
            stores_wait(1 - p)
            gathers_start(r + 1, 1 - p)
        return ()

    lax.fori_loop(0, ROUNDS // 2 - 1, body, (), unroll=False)

    gathers_wait(1)
    stores_start(ROUNDS - 1, 1)
    stores_wait(0)
    stores_wait(1)


def kernel(input_ids, table):
    idx = jnp.transpose(input_ids).reshape(-1)
    out = _sc_gather(idx, table)
    return out.reshape(SEQ, BATCH, DIM).transpose(1, 0, 2)


# speedup vs baseline: 1.0106x; 1.0106x over previous
import functools

import jax
import jax.numpy as jnp
from jax import lax
from jax.experimental import pallas as pl
from jax.experimental.pallas import tpu as pltpu
from jax.experimental.pallas import tpu_sc as plsc

BATCH = 1024
SEQ = 77
VOCAB = 49408
DIM = 768

NC = 2
NS = 16
NW = NC * NS

B = BATCH * SEQ
B_PER_W = B // NW
CHUNK = 8
NCHUNK = B_PER_W // CHUNK
K = 14
L = 7
MAIN = (NCHUNK - 2 * L) // K

_mesh = plsc.VectorSubcoreMesh(
    core_axis_name="c", subcore_axis_name="s", num_cores=NC, num_subcores=NS
)


@functools.partial(
    pl.kernel,
    mesh=_mesh,
    out_type=jax.ShapeDtypeStruct((B, DIM), jnp.float32),
    scratch_types=[
        pltpu.VMEM((B_PER_W,), jnp.int32),
        pltpu.VMEM((K, CHUNK, DIM), jnp.float32),
        pltpu.SemaphoreType.DMA((K,)),
        pltpu.SemaphoreType.DMA((K,)),
    ],
)
def _sc_gather(idx_hbm, table_hbm, out_hbm, idx_v, bufs, gsem, ssem):
    wid = lax.axis_index("s") * NC + lax.axis_index("c")
    base = wid * B_PER_W

    pltpu.sync_copy(idx_hbm.at[pl.ds(base, B_PER_W)], idx_v)

    def gather_start(c, slot):
        pltpu.async_copy(
            table_hbm.at[idx_v.at[pl.ds(c * CHUNK, CHUNK)]],
            bufs.at[slot],
            gsem.at[slot],
        )

    def gather_wait(slot):
        pltpu.make_async_copy(
            table_hbm.at[idx_v.at[pl.ds(0, CHUNK)]], bufs.at[slot], gsem.at[slot]
        ).wait()

    def store_start(c, slot):
        pltpu.async_copy(
            bufs.at[slot], out_hbm.at[pl.ds(base + c * CHUNK, CHUNK)], ssem.at[slot]
        )

    def store_wait(slot):
        pltpu.make_async_copy(
            bufs.at[slot], out_hbm.at[pl.ds(base, CHUNK)], ssem.at[slot]
        ).wait()

    for j in range(L):
        gather_start(j, j)
    for c in range(L):
        gather_start(c + L, c + L)
        gather_wait(c)
        store_start(c, c)

    def body(i, _):
        for b in range(K):
            c = L + K * i + b
            slot = (L + b) % K
            store_wait(b)
            gather_start(c + L, b)
            gather_wait(slot)
            store_start(c, slot)
        return ()

    lax.fori_loop(0, MAIN, body, (), unroll=False)

    for k in range(L):
        c = NCHUNK - L + k
        slot = c % K
        gather_wait(slot)
        store_start(c, slot)
    for b in range(K):
        store_wait(b)


def kernel(input_ids, table):
    idx = jnp.transpose(input_ids).reshape(-1)
    out = _sc_gather(idx, table)
    return out.reshape(SEQ, BATCH, DIM).transpose(1, 0, 2)
